# baseline (speedup 1.0000x reference)
import jax
import jax.numpy as jnp
from jax.experimental import pallas as pl

_K = 1000
_KP = 1024
_D = 80
_TB = 1024


def _prep_kernel(cb_ref, cbn_ref):
    cb = cb_ref[...]
    n = jnp.sqrt(jnp.sum(cb * cb, axis=-1, keepdims=True))
    cbn_ref[...] = (cb / jnp.clip(n, 1e-12, None)).astype(jnp.bfloat16)


def _label_kernel(x_ref, rpt_ref, cbn_ref, o_ref):
    xt = x_ref[...].T
    mu = jnp.mean(xt, axis=0, keepdims=True)
    xc = xt - mu
    var = jnp.mean(xc * xc, axis=0, keepdims=True)
    xn = xc / jnp.sqrt(var + 1e-5)
    projt = jax.lax.dot_general(
        rpt_ref[...], xn.astype(jnp.bfloat16),
        (((1,), (0,)), ((), ())), preferred_element_type=jnp.float32)
    pnt = (projt / jnp.clip(
        jnp.sqrt(jnp.sum(projt * projt, axis=0, keepdims=True)), 1e-12, None)
    ).astype(jnp.bfloat16)
    simt = jax.lax.dot_general(
        cbn_ref[...], pnt,
        (((1,), (0,)), ((), ())), preferred_element_type=jnp.float32)
    row = jax.lax.broadcasted_iota(jnp.int32, simt.shape, 0)
    simt = jnp.where(row < _K, simt, -jnp.inf)
    o_ref[0, 0, :] = jnp.argmax(simt, axis=0).astype(jnp.int32)


def kernel(features, rand_proj, codebook):
    B, T, D = features.shape
    K, E = codebook.shape
    cb_pad = jnp.pad(codebook, ((0, _KP - K), (0, 0)))
    cbn = pl.pallas_call(
        _prep_kernel,
        out_shape=jax.ShapeDtypeStruct((_KP, E), jnp.bfloat16),
    )(cb_pad)
    rpt_bf = rand_proj.T.astype(jnp.bfloat16)
    N = B * T
    nb = N // _TB
    xf = features.reshape(N, D)
    out = pl.pallas_call(
        _label_kernel,
        grid=(nb,),
        in_specs=[pl.BlockSpec((_TB, D), lambda i: (i, 0)),
                  pl.BlockSpec((E, D), lambda i: (0, 0)),
                  pl.BlockSpec((_KP, E), lambda i: (0, 0))],
        out_specs=pl.BlockSpec((1, 1, _TB), lambda i: (i, 0, 0)),
        out_shape=jax.ShapeDtypeStruct((nb, 1, _TB), jnp.int32),
    )(xf, rpt_bf, cbn)
    return out.reshape(B, T)


# speedup vs baseline: 1.0308x; 1.0308x over previous
import jax
import jax.numpy as jnp
from jax.experimental import pallas as pl

_K = 1000
_KP = 1024
_D = 80
_TB = 1024


def _prep_kernel(cb_ref, cbn_ref):
    cb = cb_ref[...]
    n = jnp.sqrt(jnp.sum(cb * cb, axis=-1, keepdims=True))
    cbn_ref[...] = (cb / jnp.clip(n, 1e-12, None)).astype(jnp.bfloat16)


def _label_kernel(x_ref, rpt_ref, cbn_ref, o_ref):
    xt = x_ref[0].T
    mu = jnp.mean(xt, axis=0, keepdims=True)
    xc = xt - mu
    var = jnp.mean(xc * xc, axis=0, keepdims=True)
    xn = xc / jnp.sqrt(var + 1e-5)
    projt = jax.lax.dot_general(
        rpt_ref[...], xn.astype(jnp.bfloat16),
        (((1,), (0,)), ((), ())), preferred_element_type=jnp.float32)
    pnt = (projt / jnp.clip(
        jnp.sqrt(jnp.sum(projt * projt, axis=0, keepdims=True)), 1e-12, None)
    ).astype(jnp.bfloat16)
    simt = jax.lax.dot_general(
        cbn_ref[...], pnt,
        (((1,), (0,)), ((), ())), preferred_element_type=jnp.float32)
    row = jax.lax.broadcasted_iota(jnp.int32, simt.shape, 0)
    simt = jnp.where(row < _K, simt, -jnp.inf)
    o_ref[0, 0, :] = jnp.argmax(simt, axis=0).astype(jnp.int32)


def kernel(features, rand_proj, codebook):
    B, T, D = features.shape
    K, E = codebook.shape
    cb_pad = jnp.pad(codebook, ((0, _KP - K), (0, 0)))
    cbn = pl.pallas_call(
        _prep_kernel,
        out_shape=jax.ShapeDtypeStruct((_KP, E), jnp.bfloat16),
    )(cb_pad)
    rpt_bf = rand_proj.T.astype(jnp.bfloat16)
    N = B * T
    nb = N // _TB
    tb_per_row = T // _TB
    out = pl.pallas_call(
        _label_kernel,
        grid=(nb,),
        in_specs=[pl.BlockSpec((1, _TB, D),
                               lambda i: (i // tb_per_row, i % tb_per_row, 0)),
                  pl.BlockSpec((E, D), lambda i: (0, 0)),
                  pl.BlockSpec((_KP, E), lambda i: (0, 0))],
        out_specs=pl.BlockSpec((1, 1, _TB), lambda i: (i, 0, 0)),
        out_shape=jax.ShapeDtypeStruct((nb, 1, _TB), jnp.int32),
    )(features, rpt_bf, cbn)
    return out.reshape(B, T)


# speedup vs baseline: 1.3151x; 1.2758x over previous
import jax
import jax.numpy as jnp
from jax.experimental import pallas as pl

_K = 1000
_KP = 1024
_D = 80
_TB = 1024


def _prep_kernel(cb_ref, cbn_ref):
    cb = cb_ref[...]
    n = jnp.sqrt(jnp.sum(cb * cb, axis=-1, keepdims=True))
    cbn_ref[...] = (cb / jnp.clip(n, 1e-12, None)).astype(jnp.bfloat16)


def _label_kernel(xt_ref, rpt_ref, cbn_ref, o_ref):
    xt = xt_ref[0]
    mu = jnp.mean(xt, axis=0, keepdims=True)
    xc = xt - mu
    var = jnp.mean(xc * xc, axis=0, keepdims=True)
    xn = xc / jnp.sqrt(var + 1e-5)
    projt = jax.lax.dot_general(
        rpt_ref[...], xn.astype(jnp.bfloat16),
        (((1,), (0,)), ((), ())), preferred_element_type=jnp.float32)
    pnt = (projt / jnp.clip(
        jnp.sqrt(jnp.sum(projt * projt, axis=0, keepdims=True)), 1e-12, None)
    ).astype(jnp.bfloat16)
    simt = jax.lax.dot_general(
        cbn_ref[...], pnt,
        (((1,), (0,)), ((), ())), preferred_element_type=jnp.float32)
    row = jax.lax.broadcasted_iota(jnp.int32, simt.shape, 0)
    simt = jnp.where(row < _K, simt, -jnp.inf)
    o_ref[0, 0, :] = jnp.argmax(simt, axis=0).astype(jnp.int32)


def kernel(features, rand_proj, codebook):
    B, T, D = features.shape
    K, E = codebook.shape
    cb_pad = jnp.pad(codebook, ((0, _KP - K), (0, 0)))
    cbn = pl.pallas_call(
        _prep_kernel,
        out_shape=jax.ShapeDtypeStruct((_KP, E), jnp.bfloat16),
    )(cb_pad)
    rpt_bf = rand_proj.T.astype(jnp.bfloat16)
    N = B * T
    nb = N // _TB
    tb_per_row = T // _TB
    xt3 = jnp.swapaxes(features, 1, 2)
    out = pl.pallas_call(
        _label_kernel,
        grid=(nb,),
        in_specs=[pl.BlockSpec((1, D, _TB),
                               lambda i: (i // tb_per_row, 0, i % tb_per_row)),
                  pl.BlockSpec((E, D), lambda i: (0, 0)),
                  pl.BlockSpec((_KP, E), lambda i: (0, 0))],
        out_specs=pl.BlockSpec((1, 1, _TB), lambda i: (i, 0, 0)),
        out_shape=jax.ShapeDtypeStruct((nb, 1, _TB), jnp.int32),
    )(xt3, rpt_bf, cbn)
    return out.reshape(B, T)


# speedup vs baseline: 1.3814x; 1.0504x over previous
import jax
import jax.numpy as jnp
from jax.experimental import pallas as pl
from jax.experimental.pallas import tpu as pltpu

_K = 1000
_KP = 1024
_D = 80
_TB = 1024


def _prep_kernel(cb_ref, cbn_ref):
    cb = cb_ref[...]
    n = jnp.sqrt(jnp.sum(cb * cb, axis=-1, keepdims=True))
    cbn_ref[...] = (cb / jnp.clip(n, 1e-12, None)).astype(jnp.bfloat16)


def _simt(xt, rpt_ref, cbn_ref):
    mu = jnp.mean(xt, axis=0, keepdims=True)
    xc = xt - mu
    var = jnp.mean(xc * xc, axis=0, keepdims=True)
    xn = xc / jnp.sqrt(var + 1e-5)
    projt = jax.lax.dot_general(
        rpt_ref[...], xn.astype(jnp.bfloat16),
        (((1,), (0,)), ((), ())), preferred_element_type=jnp.float32)
    pnt = (projt / jnp.clip(
        jnp.sqrt(jnp.sum(projt * projt, axis=0, keepdims=True)), 1e-12,
        None)).astype(jnp.bfloat16)
    return jax.lax.dot_general(
        cbn_ref[...], pnt,
        (((1,), (0,)), ((), ())), preferred_element_type=jnp.float32)


def _amax(simt):
    return jnp.argmax(simt[:_K], axis=0).astype(jnp.int32)


def _label_kernel(xt_ref, rpt_ref, cbn_ref, o_ref, scr_a, scr_b):
    scr_a[...] = _simt(xt_ref[0, :, :_TB], rpt_ref, cbn_ref)
    o_ref[0, 0, :] = _amax(scr_b[...])
    scr_b[...] = _simt(xt_ref[0, :, _TB:], rpt_ref, cbn_ref)
    o_ref[1, 0, :] = _amax(scr_a[...])


def kernel(features, rand_proj, codebook):
    B, T, D = features.shape
    K, E = codebook.shape
    cb_pad = jnp.pad(codebook, ((0, _KP - K), (0, 0)))
    cbn = pl.pallas_call(
        _prep_kernel,
        out_shape=jax.ShapeDtypeStruct((_KP, E), jnp.bfloat16),
    )(cb_pad)
    rpt_bf = rand_proj.T.astype(jnp.bfloat16)
    N = B * T
    nb = N // _TB
    npairs = nb // 2
    pairs_per_row = T // (2 * _TB)
    xt3 = jnp.swapaxes(features, 1, 2)

    def _x_idx(j):
        p = jnp.minimum(j, npairs - 1)
        return (p // pairs_per_row, 0, p % pairs_per_row)

    out = pl.pallas_call(
        _label_kernel,
        grid=(npairs + 1,),
        in_specs=[pl.BlockSpec((1, D, 2 * _TB), _x_idx),
                  pl.BlockSpec((E, D), lambda j: (0, 0)),
                  pl.BlockSpec((_KP, E), lambda j: (0, 0))],
        out_specs=pl.BlockSpec((2, 1, _TB), lambda j: (j, 0, 0)),
        out_shape=jax.ShapeDtypeStruct((nb + 2, 1, _TB), jnp.int32),
        scratch_shapes=[pltpu.VMEM((_KP, _TB), jnp.float32),
                        pltpu.VMEM((_KP, _TB), jnp.float32)],
    )(xt3, rpt_bf, cbn)
    return out[1:nb + 1].reshape(B, T)


# speedup vs baseline: 1.4252x; 1.0317x over previous
import jax
import jax.numpy as jnp
from jax.experimental import pallas as pl
from jax.experimental.pallas import tpu as pltpu

_K = 1000
_KP = 1024
_D = 80
_TB = 1024


def _prep_kernel(cb_ref, rp_ref, cbn_ref, rpt_ref):
    cb = cb_ref[...]
    n = jnp.sqrt(jnp.sum(cb * cb, axis=-1, keepdims=True))
    cbn_ref[:_K] = (cb / jnp.clip(n, 1e-12, None)).astype(jnp.bfloat16)
    cbn_ref[_K:] = jnp.zeros((_KP - _K, cb.shape[1]), jnp.bfloat16)
    rpt_ref[...] = rp_ref[...].T.astype(jnp.bfloat16)


def _simt(xt, rpt_ref, cbn_ref):
    mu = jnp.mean(xt, axis=0, keepdims=True)
    xc = xt - mu
    var = jnp.mean(xc * xc, axis=0, keepdims=True)
    xn = xc / jnp.sqrt(var + 1e-5)
    projt = jax.lax.dot_general(
        rpt_ref[...], xn.astype(jnp.bfloat16),
        (((1,), (0,)), ((), ())), preferred_element_type=jnp.float32)
    pnt = (projt / jnp.clip(
        jnp.sqrt(jnp.sum(projt * projt, axis=0, keepdims=True)), 1e-12,
        None)).astype(jnp.bfloat16)
    return jax.lax.dot_general(
        cbn_ref[...], pnt,
        (((1,), (0,)), ((), ())), preferred_element_type=jnp.float32)


def _amax(simt):
    return jnp.argmax(simt[:_K], axis=0).astype(jnp.int32)


def _label_kernel(xt_ref, rpt_ref, cbn_ref, o_ref, scr_a, scr_b):
    o_ref[0, 0, :] = _amax(scr_b[...])
    scr_a[...] = _simt(xt_ref[0, :, :_TB], rpt_ref, cbn_ref)
    o_ref[1, 0, :] = _amax(scr_a[...])
    scr_b[...] = _simt(xt_ref[0, :, _TB:], rpt_ref, cbn_ref)


def kernel(features, rand_proj, codebook):
    B, T, D = features.shape
    K, E = codebook.shape
    cbn, rpt_bf = pl.pallas_call(
        _prep_kernel,
        out_shape=(jax.ShapeDtypeStruct((_KP, E), jnp.bfloat16),
                   jax.ShapeDtypeStruct((E, D), jnp.bfloat16)),
    )(codebook, rand_proj)
    N = B * T
    nb = N // _TB
    npairs = nb // 2
    pairs_per_row = T // (2 * _TB)
    xt3 = jnp.swapaxes(features, 1, 2)

    def _x_idx(j):
        p = jnp.minimum(j, npairs - 1)
        return (p // pairs_per_row, 0, p % pairs_per_row)

    out = pl.pallas_call(
        _label_kernel,
        grid=(npairs + 1,),
        in_specs=[pl.BlockSpec((1, D, 2 * _TB), _x_idx),
                  pl.BlockSpec((E, D), lambda j: (0, 0)),
                  pl.BlockSpec((_KP, E), lambda j: (0, 0))],
        out_specs=pl.BlockSpec((2, 1, _TB), lambda j: (j, 0, 0)),
        out_shape=jax.ShapeDtypeStruct((nb + 2, 1, _TB), jnp.int32),
        scratch_shapes=[pltpu.VMEM((_KP, _TB), jnp.float32),
                        pltpu.VMEM((_KP, _TB), jnp.float32)],
    )(xt3, rpt_bf, cbn)
    return out[1:nb + 1].reshape(B, T)


# speedup vs baseline: 1.4634x; 1.0268x over previous
import jax
import jax.numpy as jnp
from jax.experimental import pallas as pl
from jax.experimental.pallas import tpu as pltpu

_K = 1000
_KP = 1024
_D = 80
_TB = 2048


def _prep_kernel(cb_ref, rp_ref, cbn_ref, rpt_ref):
    cb = cb_ref[...]
    n = jnp.sqrt(jnp.sum(cb * cb, axis=-1, keepdims=True))
    cbn_ref[:_K] = (cb / jnp.clip(n, 1e-12, None)).astype(jnp.bfloat16)
    cbn_ref[_K:] = jnp.zeros((_KP - _K, cb.shape[1]), jnp.bfloat16)
    rpt_ref[...] = rp_ref[...].T.astype(jnp.bfloat16)


def _simt(xt, rpt_ref, cbn_ref):
    mu = jnp.mean(xt, axis=0, keepdims=True)
    xc = xt - mu
    var = jnp.mean(xc * xc, axis=0, keepdims=True)
    xn = xc / jnp.sqrt(var + 1e-5)
    projt = jax.lax.dot_general(
        rpt_ref[...], xn.astype(jnp.bfloat16),
        (((1,), (0,)), ((), ())), preferred_element_type=jnp.float32)
    pnt = (projt / jnp.clip(
        jnp.sqrt(jnp.sum(projt * projt, axis=0, keepdims=True)), 1e-12,
        None)).astype(jnp.bfloat16)
    return jax.lax.dot_general(
        cbn_ref[...], pnt,
        (((1,), (0,)), ((), ())), preferred_element_type=jnp.float32)


def _amax(simt):
    return jnp.argmax(simt[:_K], axis=0).astype(jnp.int32)


def _label_kernel(xt_ref, rpt_ref, cbn_ref, o_ref, scr_a, scr_b):
    o_ref[0, 0, :] = _amax(scr_b[...])
    scr_a[...] = _simt(xt_ref[0, :, :_TB], rpt_ref, cbn_ref)
    o_ref[1, 0, :] = _amax(scr_a[...])
    scr_b[...] = _simt(xt_ref[0, :, _TB:], rpt_ref, cbn_ref)


def kernel(features, rand_proj, codebook):
    B, T, D = features.shape
    K, E = codebook.shape
    cbn, rpt_bf = pl.pallas_call(
        _prep_kernel,
        out_shape=(jax.ShapeDtypeStruct((_KP, E), jnp.bfloat16),
                   jax.ShapeDtypeStruct((E, D), jnp.bfloat16)),
    )(codebook, rand_proj)
    N = B * T
    nb = N // _TB
    npairs = nb // 2
    pairs_per_row = T // (2 * _TB)
    xt3 = jnp.swapaxes(features, 1, 2)

    def _x_idx(j):
        p = jnp.minimum(j, npairs - 1)
        return (p // pairs_per_row, 0, p % pairs_per_row)

    out = pl.pallas_call(
        _label_kernel,
        grid=(npairs + 1,),
        in_specs=[pl.BlockSpec((1, D, 2 * _TB), _x_idx),
                  pl.BlockSpec((E, D), lambda j: (0, 0)),
                  pl.BlockSpec((_KP, E), lambda j: (0, 0))],
        out_specs=pl.BlockSpec((2, 1, _TB), lambda j: (j, 0, 0)),
        out_shape=jax.ShapeDtypeStruct((nb + 2, 1, _TB), jnp.int32),
        scratch_shapes=[pltpu.VMEM((_KP, _TB), jnp.float32),
                        pltpu.VMEM((_KP, _TB), jnp.float32)],
    )(xt3, rpt_bf, cbn)
    return out[1:nb + 1].reshape(B, T)
